# baseline (speedup 1.0000x reference)
import jax
import jax.numpy as jnp
from jax import lax
from jax.experimental import pallas as pl
from jax.experimental.pallas import tpu as pltpu

_B, _S, _D, _H = 4, 2048, 1024, 1024
_MT = 512
_TPR = _S // _MT
_NT = _B * _TPR
_LL = 128
_PS = _MT // _LL


def _body(emb_ref, w1_ref, b1_ref, w2_ref, b2_ref,
          filt_ref, mask_ref, ek_ref,
          embscr, lgscr, lgscr_p, thr_scr, pi_scr):
    s = pl.program_id(0)
    b = s // (2 * _TPR)
    ph = (s // _TPR) % 2
    t = s % _TPR
    tile = b * _TPR + t

    @pl.when(ph == 0)
    def _score():
        x = jnp.dot(emb_ref[...], w1_ref[...], preferred_element_type=jnp.float32)
        x = jnp.maximum(x + b1_ref[...], 0.0)
        lg = jnp.dot(x, w2_ref[...], preferred_element_type=jnp.float32)
        lg = lg + b2_ref[...]
        embscr[pl.ds(t * _MT, _MT), :] = emb_ref[...]
        lgscr_p[pl.ds(t * _MT, _MT), :] = lg
        tmask = lax.broadcasted_iota(jnp.int32, (_NT, _MT), 0) == tile
        lgT = jnp.broadcast_to(jnp.swapaxes(lg, 0, 1), (_NT, _MT))
        lgscr[...] = jnp.where(tmask, lgT, lgscr[...])

    @pl.when((ph == 0) & (t == _TPR - 1))
    def _row_epilogue():
        lgs = lgscr[...]
        rl = (lax.broadcasted_iota(jnp.int32, (_NT, _MT), 0) // _TPR) == b
        ek = jnp.sum(jnp.where(rl, jax.nn.sigmoid(lgs), 0.0),
                     axis=(0, 1), keepdims=True)
        bm = lax.broadcasted_iota(jnp.int32, (_B, 1), 0) == b
        ek_ref[...] = jnp.where(bm, jnp.broadcast_to(ek, (_B, 1)), ek_ref[...])
        k = jnp.maximum(ek.astype(jnp.int32), 32)

        bits = lax.bitcast_convert_type(lgs, jnp.int32)
        key = jnp.where(bits < 0, bits ^ jnp.int32(0x7FFFFFFF), bits)

        def rowcount(pred):
            return jnp.sum(jnp.where(rl & pred, 1, 0),
                           axis=(0, 1), keepdims=True)

        nonneg = key >= 0
        cnt_nn = rowcount(nonneg)
        in_pos = k <= cnt_nn
        kk = jnp.where(in_pos, k, k - cnt_nn)
        cls = nonneg == in_pos
        m = key & jnp.int32(0x7FFFFFFF)
        p = jnp.zeros_like(k)
        for b_idx in range(30, -1, -1):
            q = p + jnp.int32(1 << b_idx)
            c = rowcount(cls & (m >= q))
            p = jnp.where(c >= kk, q, p)
        thr = jnp.where(in_pos, p, p | jnp.int32(-2147483648))

        c_gt = rowcount(key > thr)
        r = k - c_gt
        tie = key == thr
        sidx = ((lax.broadcasted_iota(jnp.int32, (_NT, _MT), 0) % _TPR) * _MT
                + lax.broadcasted_iota(jnp.int32, (_NT, _MT), 1))
        pi = jnp.zeros_like(k)
        for b_idx in range(11, -1, -1):
            qi = pi + jnp.int32(1 << b_idx)
            ci = rowcount(tie & (sidx < qi))
            pi = jnp.where(ci < r, qi, pi)

        thr_scr[...] = jnp.where(bm, jnp.broadcast_to(thr, (_B, 1)),
                                 thr_scr[...])
        pi_scr[...] = jnp.where(bm, jnp.broadcast_to(pi, (_B, 1)),
                                pi_scr[...])

    @pl.when(ph == 1)
    def _emit():
        bm = lax.broadcasted_iota(jnp.int32, (_B, 1), 0) == b
        thr = jnp.sum(jnp.where(bm, thr_scr[...], 0),
                      axis=(0, 1), keepdims=True)
        pi = jnp.sum(jnp.where(bm, pi_scr[...], 0),
                     axis=(0, 1), keepdims=True)
        lg = lgscr_p[pl.ds(t * _MT, _MT), :]
        bits = lax.bitcast_convert_type(lg, jnp.int32)
        key = jnp.where(bits < 0, bits ^ jnp.int32(0x7FFFFFFF), bits)
        sidx = t * _MT + lax.broadcasted_iota(jnp.int32, (_MT, 1), 0)
        hard = (key > thr) | ((key == thr) & (sidx <= pi))
        mk = hard.astype(jnp.float32)
        mask_ref[...] = mk
        filt_ref[...] = embscr[pl.ds(t * _MT, _MT), :] * mk


def kernel(token_embeddings, W1, b1, W2, b2):
    emb2d = token_embeddings.reshape(_B * _S, _D)

    def in_idx(s):
        b = s // (2 * _TPR)
        ph = (s // _TPR) % 2
        t = s % _TPR
        tile = b * _TPR + t
        return (jnp.where(ph == 0, tile, jnp.minimum(tile + _TPR, _NT - 1)), 0)

    def out_idx(s):
        b = s // (2 * _TPR)
        ph = (s // _TPR) % 2
        t = s % _TPR
        tile = b * _TPR + t
        return (jnp.where(ph == 0, jnp.maximum(b * _TPR - 1, 0), tile), 0)

    filt, mask, ekv = pl.pallas_call(
        _body,
        grid=(2 * _NT,),
        in_specs=[
            pl.BlockSpec((_MT, _D), in_idx),
            pl.BlockSpec((_D, _H), lambda s: (0, 0)),
            pl.BlockSpec((1, _H), lambda s: (0, 0)),
            pl.BlockSpec((_H, 1), lambda s: (0, 0)),
            pl.BlockSpec((1, 1), lambda s: (0, 0)),
        ],
        out_specs=(
            pl.BlockSpec((_MT, _D), out_idx),
            pl.BlockSpec((_MT, 1), out_idx),
            pl.BlockSpec((_B, 1), lambda s: (0, 0)),
        ),
        out_shape=(
            jax.ShapeDtypeStruct((_B * _S, _D), jnp.float32),
            jax.ShapeDtypeStruct((_B * _S, 1), jnp.float32),
            jax.ShapeDtypeStruct((_B, 1), jnp.float32),
        ),
        scratch_shapes=[
            pltpu.VMEM((_S, _D), jnp.float32),
            pltpu.VMEM((_NT, _MT), jnp.float32),
            pltpu.VMEM((_S, 1), jnp.float32),
            pltpu.VMEM((_B, 1), jnp.int32),
            pltpu.VMEM((_B, 1), jnp.int32),
        ],
        compiler_params=pltpu.CompilerParams(
            dimension_semantics=("arbitrary",),
        ),
    )(emb2d, W1, b1.reshape(1, _H), W2, b2.reshape(1, 1))

    return filt.reshape(_B, _S, _D), mask.reshape(_B, _S), ekv[:, 0]


# speedup vs baseline: 1.0703x; 1.0703x over previous
import jax
import jax.numpy as jnp
from jax import lax
from jax.experimental import pallas as pl
from jax.experimental.pallas import tpu as pltpu

_B, _S, _D, _H = 4, 2048, 1024, 1024
_MT = 512
_TPR = _S // _MT
_NT = _B * _TPR
_LL = 128
_PS = _MT // _LL


def _body(emb_ref, w1_ref, b1_ref, w2_ref, b2_ref,
          filt_ref, mask_ref, ek_ref,
          embscr, lgscr, lgscr_p, thr_scr, pi_scr):
    s = pl.program_id(0)
    b = s // (2 * _TPR)
    ph = (s // _TPR) % 2
    t = s % _TPR
    tile = b * _TPR + t

    @pl.when(ph == 0)
    def _score():
        x = jnp.dot(emb_ref[...], w1_ref[...], preferred_element_type=jnp.float32)
        x = jnp.maximum(x + b1_ref[...], 0.0)
        lg = jnp.dot(x, w2_ref[...], preferred_element_type=jnp.float32)
        lg = lg + b2_ref[...]
        embscr[pl.ds(t * _MT, _MT), :] = emb_ref[...]
        lgscr_p[pl.ds(t * _MT, _MT), :] = lg
        tmask = lax.broadcasted_iota(jnp.int32, (_NT, _MT), 0) == tile
        lgT = jnp.broadcast_to(jnp.swapaxes(lg, 0, 1), (_NT, _MT))
        lgscr[...] = jnp.where(tmask, lgT, lgscr[...])

    @pl.when((ph == 0) & (t == _TPR - 1))
    def _row_epilogue():
        lgs = lgscr[...]
        rl = (lax.broadcasted_iota(jnp.int32, (_NT, _MT), 0) // _TPR) == b
        ek = jnp.sum(jnp.where(rl, jax.nn.sigmoid(lgs), 0.0),
                     axis=(0, 1), keepdims=True)
        bm = lax.broadcasted_iota(jnp.int32, (_B, 1), 0) == b
        ek_ref[...] = jnp.where(bm, jnp.broadcast_to(ek, (_B, 1)), ek_ref[...])
        k = jnp.maximum(ek.astype(jnp.int32), 32)

        bits = lax.bitcast_convert_type(lgs, jnp.int32)
        key = jnp.where(bits < 0, bits ^ jnp.int32(0x7FFFFFFF), bits)

        def rowcount(pred):
            return jnp.sum(jnp.where(rl & pred, 1, 0),
                           axis=(0, 1), keepdims=True)

        nonneg = key >= 0
        cnt_nn = rowcount(nonneg)
        in_pos = k <= cnt_nn
        kk = jnp.where(in_pos, k, k - cnt_nn)
        cls = nonneg == in_pos
        m = key & jnp.int32(0x7FFFFFFF)
        p = jnp.zeros_like(k)
        for b_idx in range(30, -1, -1):
            q = p + jnp.int32(1 << b_idx)
            c = rowcount(cls & (m >= q))
            p = jnp.where(c >= kk, q, p)
        thr = jnp.where(in_pos, p, p | jnp.int32(-2147483648))

        c_gt = rowcount(key > thr)
        r = k - c_gt
        tie = key == thr
        sidx = ((lax.broadcasted_iota(jnp.int32, (_NT, _MT), 0) % _TPR) * _MT
                + lax.broadcasted_iota(jnp.int32, (_NT, _MT), 1))
        pi = jnp.zeros_like(k)
        for b_idx in range(11, -1, -1):
            qi = pi + jnp.int32(1 << b_idx)
            ci = rowcount(tie & (sidx < qi))
            pi = jnp.where(ci < r, qi, pi)

        thr_scr[...] = jnp.where(bm, jnp.broadcast_to(thr, (_B, 1)),
                                 thr_scr[...])
        pi_scr[...] = jnp.where(bm, jnp.broadcast_to(pi, (_B, 1)),
                                pi_scr[...])

    @pl.when(ph == 1)
    def _emit():
        bm = lax.broadcasted_iota(jnp.int32, (_B, 1), 0) == b
        thr = jnp.sum(jnp.where(bm, thr_scr[...], 0),
                      axis=(0, 1), keepdims=True)
        pi = jnp.sum(jnp.where(bm, pi_scr[...], 0),
                     axis=(0, 1), keepdims=True)
        lg = lgscr_p[pl.ds(t * _MT, _MT), :]
        bits = lax.bitcast_convert_type(lg, jnp.int32)
        key = jnp.where(bits < 0, bits ^ jnp.int32(0x7FFFFFFF), bits)
        sidx = t * _MT + lax.broadcasted_iota(jnp.int32, (_MT, 1), 0)
        hard = (key > thr) | ((key == thr) & (sidx <= pi))
        mk = hard.astype(jnp.float32)
        mask_ref[...] = mk
        filt_ref[...] = embscr[pl.ds(t * _MT, _MT), :] * mk


def kernel(token_embeddings, W1, b1, W2, b2):
    emb2d = token_embeddings.reshape(_B * _S, _D)

    def in_idx(s):
        b = s // (2 * _TPR)
        ph = (s // _TPR) % 2
        t = s % _TPR
        tile = b * _TPR + t
        return (jnp.where(ph == 0, tile, b * _TPR + _TPR - 1), 0)

    def out_idx(s):
        b = s // (2 * _TPR)
        ph = (s // _TPR) % 2
        t = s % _TPR
        tile = b * _TPR + t
        return (jnp.where(ph == 0, jnp.maximum(b * _TPR - 1, 0), tile), 0)

    filt, mask, ekv = pl.pallas_call(
        _body,
        grid=(2 * _NT,),
        in_specs=[
            pl.BlockSpec((_MT, _D), in_idx),
            pl.BlockSpec((_D, _H), lambda s: (0, 0)),
            pl.BlockSpec((1, _H), lambda s: (0, 0)),
            pl.BlockSpec((_H, 1), lambda s: (0, 0)),
            pl.BlockSpec((1, 1), lambda s: (0, 0)),
        ],
        out_specs=(
            pl.BlockSpec((_MT, _D), out_idx),
            pl.BlockSpec((_MT, 1), out_idx),
            pl.BlockSpec((_B, 1), lambda s: (0, 0)),
        ),
        out_shape=(
            jax.ShapeDtypeStruct((_B * _S, _D), jnp.float32),
            jax.ShapeDtypeStruct((_B * _S, 1), jnp.float32),
            jax.ShapeDtypeStruct((_B, 1), jnp.float32),
        ),
        scratch_shapes=[
            pltpu.VMEM((_S, _D), jnp.float32),
            pltpu.VMEM((_NT, _MT), jnp.float32),
            pltpu.VMEM((_S, 1), jnp.float32),
            pltpu.VMEM((_B, 1), jnp.int32),
            pltpu.VMEM((_B, 1), jnp.int32),
        ],
        compiler_params=pltpu.CompilerParams(
            dimension_semantics=("arbitrary",),
        ),
    )(emb2d, W1, b1.reshape(1, _H), W2, b2.reshape(1, 1))

    return filt.reshape(_B, _S, _D), mask.reshape(_B, _S), ekv[:, 0]


# speedup vs baseline: 1.1874x; 1.1094x over previous
import jax
import jax.numpy as jnp
from jax import lax
from jax.experimental import pallas as pl

_B, _S, _D, _H = 4, 2048, 1024, 1024
_MT = 512
_NT = (_B * _S) // _MT


def _logits_body(emb_ref, w1_ref, b1_ref, w2_ref, b2_ref, out_ref):
    x = jnp.dot(emb_ref[...], w1_ref[...], preferred_element_type=jnp.float32)
    x = jnp.maximum(x + b1_ref[...], 0.0)
    lg = jnp.dot(x, w2_ref[...], preferred_element_type=jnp.float32)
    out_ref[...] = lg[:, 0:1] + b2_ref[...]


def _mask_body(lg_ref, mask_ref, ek_ref):
    lg = lg_ref[...]
    ek = jnp.sum(jax.nn.sigmoid(lg), axis=1, keepdims=True)
    ek_ref[...] = ek
    k = jnp.maximum(ek.astype(jnp.int32), 32)

    bits = lax.bitcast_convert_type(lg, jnp.int32)
    key = jnp.where(bits < 0, bits ^ jnp.int32(0x7FFFFFFF), bits)

    nonneg = key >= 0
    cnt_nn = jnp.sum(nonneg.astype(jnp.int32), axis=1, keepdims=True)
    in_pos = k <= cnt_nn
    kk = jnp.where(in_pos, k, k - cnt_nn)
    cls = nonneg == in_pos
    m = key & jnp.int32(0x7FFFFFFF)
    p = jnp.zeros_like(k)
    for b_idx in range(30, -1, -1):
        q = p + jnp.int32(1 << b_idx)
        c = jnp.sum(jnp.where(cls & (m >= q), 1, 0), axis=1, keepdims=True)
        p = jnp.where(c >= kk, q, p)
    thr = jnp.where(in_pos, p, p | jnp.int32(-2147483648))

    gt = key > thr
    c_gt = jnp.sum(gt.astype(jnp.int32), axis=1, keepdims=True)
    r = k - c_gt
    tie = key == thr
    idx = lax.broadcasted_iota(jnp.int32, (_B, _S), 1)
    pi = jnp.zeros_like(k)
    for b_idx in range(11, -1, -1):
        qi = pi + jnp.int32(1 << b_idx)
        ci = jnp.sum(jnp.where(tie & (idx < qi), 1, 0), axis=1, keepdims=True)
        pi = jnp.where(ci < r, qi, pi)
    hard = gt | (tie & (idx <= pi))
    mask_ref[...] = hard.astype(jnp.float32)


def _filter_body(emb_ref, mk_ref, out_ref):
    out_ref[...] = emb_ref[...] * mk_ref[...]


def kernel(token_embeddings, W1, b1, W2, b2):
    emb2d = token_embeddings.reshape(_B * _S, _D)
    logits_col = pl.pallas_call(
        _logits_body,
        grid=(_NT,),
        in_specs=[
            pl.BlockSpec((_MT, _D), lambda i: (i, 0)),
            pl.BlockSpec((_D, _H), lambda i: (0, 0)),
            pl.BlockSpec((1, _H), lambda i: (0, 0)),
            pl.BlockSpec((_D, 1), lambda i: (0, 0)),
            pl.BlockSpec((1, 1), lambda i: (0, 0)),
        ],
        out_specs=pl.BlockSpec((_MT, 1), lambda i: (i, 0)),
        out_shape=jax.ShapeDtypeStruct((_B * _S, 1), jnp.float32),
    )(emb2d, W1, b1.reshape(1, _H), W2, b2.reshape(1, 1))
    logits = logits_col.reshape(_B, _S)

    mask, ek = pl.pallas_call(
        _mask_body,
        out_shape=(
            jax.ShapeDtypeStruct((_B, _S), jnp.float32),
            jax.ShapeDtypeStruct((_B, 1), jnp.float32),
        ),
    )(logits)

    filt = pl.pallas_call(
        _filter_body,
        grid=(_NT,),
        in_specs=[
            pl.BlockSpec((_MT, _D), lambda i: (i, 0)),
            pl.BlockSpec((_MT, 1), lambda i: (i, 0)),
        ],
        out_specs=pl.BlockSpec((_MT, _D), lambda i: (i, 0)),
        out_shape=jax.ShapeDtypeStruct((_B * _S, _D), jnp.float32),
    )(emb2d, mask.reshape(_B * _S, 1))

    return filt.reshape(_B, _S, _D), mask, ek.reshape(_B)


# speedup vs baseline: 1.2718x; 1.0711x over previous
import jax
import jax.numpy as jnp
from jax import lax
from jax.experimental import pallas as pl

_B, _S, _D, _H = 4, 2048, 1024, 1024
_MT = 1024
_NT = (_B * _S) // _MT


def _logits_body(emb_ref, w1_ref, b1_ref, w2_ref, b2_ref, out_ref):
    x = jnp.dot(emb_ref[...], w1_ref[...], preferred_element_type=jnp.float32)
    x = jnp.maximum(x + b1_ref[...], 0.0)
    lg = jnp.dot(x, w2_ref[...], preferred_element_type=jnp.float32)
    out_ref[...] = lg[:, 0:1] + b2_ref[...]


def _mask_body(lg_ref, mask_ref, ek_ref):
    lg = lg_ref[...]
    ek = jnp.sum(jax.nn.sigmoid(lg), axis=1, keepdims=True)
    ek_ref[...] = ek
    k = jnp.maximum(ek.astype(jnp.int32), 32)

    bits = lax.bitcast_convert_type(lg, jnp.int32)
    key = jnp.where(bits < 0, bits ^ jnp.int32(0x7FFFFFFF), bits)

    nonneg = key >= 0
    cnt_nn = jnp.sum(nonneg.astype(jnp.int32), axis=1, keepdims=True)
    in_pos = k <= cnt_nn
    kk = jnp.where(in_pos, k, k - cnt_nn)
    cls = nonneg == in_pos
    m = key & jnp.int32(0x7FFFFFFF)
    p = jnp.zeros_like(k)
    for b_idx in range(30, -1, -1):
        q = p + jnp.int32(1 << b_idx)
        c = jnp.sum(jnp.where(cls & (m >= q), 1, 0), axis=1, keepdims=True)
        p = jnp.where(c >= kk, q, p)
    thr = jnp.where(in_pos, p, p | jnp.int32(-2147483648))

    gt = key > thr
    c_gt = jnp.sum(gt.astype(jnp.int32), axis=1, keepdims=True)
    r = k - c_gt
    tie = key == thr
    idx = lax.broadcasted_iota(jnp.int32, (_B, _S), 1)
    pi = jnp.zeros_like(k)
    for b_idx in range(11, -1, -1):
        qi = pi + jnp.int32(1 << b_idx)
        ci = jnp.sum(jnp.where(tie & (idx < qi), 1, 0), axis=1, keepdims=True)
        pi = jnp.where(ci < r, qi, pi)
    hard = gt | (tie & (idx <= pi))
    mask_ref[...] = hard.astype(jnp.float32)


def _filter_body(emb_ref, mk_ref, out_ref):
    out_ref[...] = emb_ref[...] * mk_ref[...]


def kernel(token_embeddings, W1, b1, W2, b2):
    emb2d = token_embeddings.reshape(_B * _S, _D)
    logits_col = pl.pallas_call(
        _logits_body,
        grid=(_NT,),
        in_specs=[
            pl.BlockSpec((_MT, _D), lambda i: (i, 0)),
            pl.BlockSpec((_D, _H), lambda i: (0, 0)),
            pl.BlockSpec((1, _H), lambda i: (0, 0)),
            pl.BlockSpec((_D, 1), lambda i: (0, 0)),
            pl.BlockSpec((1, 1), lambda i: (0, 0)),
        ],
        out_specs=pl.BlockSpec((_MT, 1), lambda i: (i, 0)),
        out_shape=jax.ShapeDtypeStruct((_B * _S, 1), jnp.float32),
    )(emb2d, W1, b1.reshape(1, _H), W2, b2.reshape(1, 1))
    logits = logits_col.reshape(_B, _S)

    mask, ek = pl.pallas_call(
        _mask_body,
        out_shape=(
            jax.ShapeDtypeStruct((_B, _S), jnp.float32),
            jax.ShapeDtypeStruct((_B, 1), jnp.float32),
        ),
    )(logits)

    filt = pl.pallas_call(
        _filter_body,
        grid=(_NT,),
        in_specs=[
            pl.BlockSpec((_MT, _D), lambda i: (i, 0)),
            pl.BlockSpec((_MT, 1), lambda i: (i, 0)),
        ],
        out_specs=pl.BlockSpec((_MT, _D), lambda i: (i, 0)),
        out_shape=jax.ShapeDtypeStruct((_B * _S, _D), jnp.float32),
    )(emb2d, mask.reshape(_B * _S, 1))

    return filt.reshape(_B, _S, _D), mask, ek.reshape(_B)


# speedup vs baseline: 1.2843x; 1.0098x over previous
import jax
import jax.numpy as jnp
from jax import lax
from jax.experimental import pallas as pl

_B, _S, _D, _H = 4, 2048, 1024, 1024
_MT = 2048
_NT = (_B * _S) // _MT


def _logits_body(emb_ref, w1_ref, b1_ref, w2_ref, b2_ref, out_ref):
    x = jnp.dot(emb_ref[...], w1_ref[...], preferred_element_type=jnp.float32)
    x = jnp.maximum(x + b1_ref[...], 0.0)
    lg = jnp.dot(x, w2_ref[...], preferred_element_type=jnp.float32)
    out_ref[...] = lg[:, 0:1] + b2_ref[...]


def _mask_body(lg_ref, mask_ref, ek_ref):
    lg = lg_ref[...]
    ek = jnp.sum(jax.nn.sigmoid(lg), axis=1, keepdims=True)
    ek_ref[...] = ek
    k = jnp.maximum(ek.astype(jnp.int32), 32)

    bits = lax.bitcast_convert_type(lg, jnp.int32)
    key = jnp.where(bits < 0, bits ^ jnp.int32(0x7FFFFFFF), bits)

    nonneg = key >= 0
    cnt_nn = jnp.sum(nonneg.astype(jnp.int32), axis=1, keepdims=True)
    in_pos = k <= cnt_nn
    kk = jnp.where(in_pos, k, k - cnt_nn)
    cls = nonneg == in_pos
    m = key & jnp.int32(0x7FFFFFFF)
    p = jnp.zeros_like(k)
    for b_idx in range(30, -1, -1):
        q = p + jnp.int32(1 << b_idx)
        c = jnp.sum(jnp.where(cls & (m >= q), 1, 0), axis=1, keepdims=True)
        p = jnp.where(c >= kk, q, p)
    thr = jnp.where(in_pos, p, p | jnp.int32(-2147483648))

    gt = key > thr
    c_gt = jnp.sum(gt.astype(jnp.int32), axis=1, keepdims=True)
    r = k - c_gt
    tie = key == thr
    idx = lax.broadcasted_iota(jnp.int32, (_B, _S), 1)
    pi = jnp.zeros_like(k)
    for b_idx in range(11, -1, -1):
        qi = pi + jnp.int32(1 << b_idx)
        ci = jnp.sum(jnp.where(tie & (idx < qi), 1, 0), axis=1, keepdims=True)
        pi = jnp.where(ci < r, qi, pi)
    hard = gt | (tie & (idx <= pi))
    mask_ref[...] = hard.astype(jnp.float32)


def _filter_body(emb_ref, mk_ref, out_ref):
    out_ref[...] = emb_ref[...] * mk_ref[...]


def kernel(token_embeddings, W1, b1, W2, b2):
    emb2d = token_embeddings.reshape(_B * _S, _D)
    logits_col = pl.pallas_call(
        _logits_body,
        grid=(_NT,),
        in_specs=[
            pl.BlockSpec((_MT, _D), lambda i: (i, 0)),
            pl.BlockSpec((_D, _H), lambda i: (0, 0)),
            pl.BlockSpec((1, _H), lambda i: (0, 0)),
            pl.BlockSpec((_D, 1), lambda i: (0, 0)),
            pl.BlockSpec((1, 1), lambda i: (0, 0)),
        ],
        out_specs=pl.BlockSpec((_MT, 1), lambda i: (i, 0)),
        out_shape=jax.ShapeDtypeStruct((_B * _S, 1), jnp.float32),
    )(emb2d, W1, b1.reshape(1, _H), W2, b2.reshape(1, 1))
    logits = logits_col.reshape(_B, _S)

    mask, ek = pl.pallas_call(
        _mask_body,
        out_shape=(
            jax.ShapeDtypeStruct((_B, _S), jnp.float32),
            jax.ShapeDtypeStruct((_B, 1), jnp.float32),
        ),
    )(logits)

    filt = pl.pallas_call(
        _filter_body,
        grid=(_NT,),
        in_specs=[
            pl.BlockSpec((_MT, _D), lambda i: (i, 0)),
            pl.BlockSpec((_MT, 1), lambda i: (i, 0)),
        ],
        out_specs=pl.BlockSpec((_MT, _D), lambda i: (i, 0)),
        out_shape=jax.ShapeDtypeStruct((_B * _S, _D), jnp.float32),
    )(emb2d, mask.reshape(_B * _S, 1))

    return filt.reshape(_B, _S, _D), mask, ek.reshape(_B)


# speedup vs baseline: 1.4102x; 1.0981x over previous
import jax
import jax.numpy as jnp
from jax import lax
from jax.experimental import pallas as pl
from jax.experimental.pallas import tpu as pltpu

_B, _S, _D, _H = 4, 2048, 1024, 1024
_MT = 2048
_NT = (_B * _S) // _MT


def _logits_body(emb_ref, w1_ref, b1_ref, w2_ref, b2_ref, out_ref):
    x = jnp.dot(emb_ref[...], w1_ref[...], preferred_element_type=jnp.float32)
    x = jnp.maximum(x + b1_ref[...], 0.0)
    lg = jnp.dot(x, w2_ref[...], preferred_element_type=jnp.float32)
    out_ref[...] = lg[:, 0:1] + b2_ref[...]


def _mask_filter_body(lg_ref, emb_ref, mask_ref, ek_ref, filt_ref,
                      mk_scr):
    i = pl.program_id(0)

    @pl.when(i == 0)
    def _select():
        lg = lg_ref[...]
        ek = jnp.sum(jax.nn.sigmoid(lg), axis=1, keepdims=True)
        ek_ref[...] = ek
        k = jnp.maximum(ek.astype(jnp.int32), 32)

        bits = lax.bitcast_convert_type(lg, jnp.int32)
        key = jnp.where(bits < 0, bits ^ jnp.int32(0x7FFFFFFF), bits)

        nonneg = key >= 0
        cnt_nn = jnp.sum(nonneg.astype(jnp.int32), axis=1, keepdims=True)
        in_pos = k <= cnt_nn
        kk = jnp.where(in_pos, k, k - cnt_nn)
        cls = nonneg == in_pos
        m = key & jnp.int32(0x7FFFFFFF)
        p = jnp.zeros_like(k)
        for b_idx in range(30, -1, -1):
            q = p + jnp.int32(1 << b_idx)
            c = jnp.sum(jnp.where(cls & (m >= q), 1, 0), axis=1, keepdims=True)
            p = jnp.where(c >= kk, q, p)
        thr = jnp.where(in_pos, p, p | jnp.int32(-2147483648))

        gt = key > thr
        c_gt = jnp.sum(gt.astype(jnp.int32), axis=1, keepdims=True)
        r = k - c_gt
        tie = key == thr
        idx = lax.broadcasted_iota(jnp.int32, (_B, _S), 1)
        pi = jnp.zeros_like(k)
        for b_idx in range(11, -1, -1):
            qi = pi + jnp.int32(1 << b_idx)
            ci = jnp.sum(jnp.where(tie & (idx < qi), 1, 0), axis=1,
                         keepdims=True)
            pi = jnp.where(ci < r, qi, pi)
        hard = gt | (tie & (idx <= pi))
        mk = hard.astype(jnp.float32)
        mask_ref[...] = mk
        mk_scr[...] = mk

    mk_row = mk_scr[pl.ds(i, 1), :]
    filt_ref[...] = emb_ref[...] * jnp.swapaxes(mk_row, 0, 1)


def kernel(token_embeddings, W1, b1, W2, b2):
    emb2d = token_embeddings.reshape(_B * _S, _D)
    logits_col = pl.pallas_call(
        _logits_body,
        grid=(_NT,),
        in_specs=[
            pl.BlockSpec((_MT, _D), lambda i: (i, 0)),
            pl.BlockSpec((_D, _H), lambda i: (0, 0)),
            pl.BlockSpec((1, _H), lambda i: (0, 0)),
            pl.BlockSpec((_D, 1), lambda i: (0, 0)),
            pl.BlockSpec((1, 1), lambda i: (0, 0)),
        ],
        out_specs=pl.BlockSpec((_MT, 1), lambda i: (i, 0)),
        out_shape=jax.ShapeDtypeStruct((_B * _S, 1), jnp.float32),
    )(emb2d, W1, b1.reshape(1, _H), W2, b2.reshape(1, 1))
    logits = logits_col.reshape(_B, _S)

    mask, ek, filt = pl.pallas_call(
        _mask_filter_body,
        grid=(_B,),
        in_specs=[
            pl.BlockSpec((_B, _S), lambda i: (0, 0)),
            pl.BlockSpec((_S, _D), lambda i: (i, 0)),
        ],
        out_specs=(
            pl.BlockSpec((_B, _S), lambda i: (0, 0)),
            pl.BlockSpec((_B, 1), lambda i: (0, 0)),
            pl.BlockSpec((_S, _D), lambda i: (i, 0)),
        ),
        out_shape=(
            jax.ShapeDtypeStruct((_B, _S), jnp.float32),
            jax.ShapeDtypeStruct((_B, 1), jnp.float32),
            jax.ShapeDtypeStruct((_B * _S, _D), jnp.float32),
        ),
        scratch_shapes=[
            pltpu.VMEM((_B, _S), jnp.float32),
        ],
        compiler_params=pltpu.CompilerParams(
            dimension_semantics=("arbitrary",),
        ),
    )(logits, emb2d)

    return filt.reshape(_B, _S, _D), mask, ek.reshape(_B)
